# baseline (speedup 1.0000x reference)
import functools

import jax
import jax.numpy as jnp
from jax import lax
from jax.experimental import pallas as pl
from jax.experimental.pallas import tpu as pltpu
from jax.experimental.pallas import tpu_sc as plsc

N_NODES = 10000
N_EDGES = 320000
D = 128

NC = 2
NS = 16
NW = NC * NS
CHUNK = 128
GROUP = 16
GROUPS_PER_TILE = 5
CHUNKS_PER_TILE = GROUP * GROUPS_PER_TILE
E_PAD = NW * CHUNKS_PER_TILE * CHUNK
N_ACC = 10112
N_JUNK = N_ACC - N_NODES
ROWS_PER_TILE = N_ACC // NS
DROWS = N_ACC // CHUNK


def _sc_body(x_hbm, src_hbm, dst_hbm,
             agg_out, deg_out,
             agg_sh, src_v, dst_v, rows_a, rows_b, deg_v,
             gsem_a, gsem_b, ssem_a, ssem_b):
    c = lax.axis_index("c")
    s = lax.axis_index("s")
    wid = c * NS + s

    zero16 = jnp.zeros((16,), jnp.float32)

    def zrow(r, carry):
        for k in range(CHUNK // 16):
            deg_v[pl.ds(r * CHUNK + k * 16, 16)] = zero16
        return carry

    lax.fori_loop(0, DROWS, zrow, 0)

    def zbuf(r, carry):
        for k in range(CHUNK // 16):
            rows_a[r, pl.ds(k * 16, 16)] = zero16
        return carry

    lax.fori_loop(0, CHUNK, zbuf, 0)
    for q in range(4):
        pltpu.sync_copy(
            rows_a,
            agg_sh.at[pl.ds(s * ROWS_PER_TILE + q * CHUNK, CHUNK)])
    pltpu.sync_copy(
        rows_a.at[pl.ds(0, ROWS_PER_TILE - 4 * CHUNK)],
        agg_sh.at[pl.ds(s * ROWS_PER_TILE + 4 * CHUNK,
                        ROWS_PER_TILE - 4 * CHUNK)])
    plsc.subcore_barrier()

    one16 = jnp.ones((16,), jnp.float32)
    rows_bufs = (rows_a, rows_b)
    gsems = (gsem_a, gsem_b)
    ssems = (ssem_a, ssem_b)

    def group(g, carry):
        pltpu.sync_copy(src_hbm.at[wid, pl.ds(g * GROUP, GROUP)], src_v)
        pltpu.sync_copy(dst_hbm.at[wid, pl.ds(g * GROUP, GROUP)], dst_v)
        gathers = [
            pltpu.async_copy(x_hbm.at[src_v.at[0]], rows_bufs[0], gsems[0]),
            pltpu.async_copy(x_hbm.at[src_v.at[1]], rows_bufs[1], gsems[1]),
        ]
        for jj in range(GROUP):
            b = jj % 2
            for k in range(CHUNK // 16):
                dk = dst_v[jj, pl.ds(k * 16, 16)]
                plsc.addupdate_scatter(deg_v, [dk], one16)
            gathers[b].wait()
            pltpu.sync_copy(rows_bufs[b], agg_sh.at[dst_v.at[jj]], add=True)
            if jj + 2 < GROUP:
                gathers[b] = pltpu.async_copy(
                    x_hbm.at[src_v.at[jj + 2]], rows_bufs[b], gsems[b])
        return carry

    lax.fori_loop(0, GROUPS_PER_TILE, group, 0)

    plsc.subcore_barrier()
    pltpu.sync_copy(agg_sh.at[pl.ds(s * ROWS_PER_TILE, ROWS_PER_TILE)],
                    agg_out.at[c, pl.ds(s * ROWS_PER_TILE, ROWS_PER_TILE)])
    pltpu.sync_copy(deg_v, deg_out.at[pl.ds(wid * N_ACC, N_ACC)])


_sc_aggregate = functools.partial(
    pl.kernel,
    out_type=[
        jax.ShapeDtypeStruct((NC, N_ACC, D), jnp.float32),
        jax.ShapeDtypeStruct((NW * N_ACC + 1024,), jnp.float32),
    ],
    mesh=plsc.VectorSubcoreMesh(core_axis_name="c", subcore_axis_name="s"),
    compiler_params=pltpu.CompilerParams(needs_layout_passes=False),
    scratch_types=[
        pltpu.VMEM_SHARED((N_ACC, D), jnp.float32),
        pltpu.VMEM((GROUP, CHUNK), jnp.int32),
        pltpu.VMEM((GROUP, CHUNK), jnp.int32),
        pltpu.VMEM((CHUNK, D), jnp.float32),
        pltpu.VMEM((CHUNK, D), jnp.float32),
        pltpu.VMEM((N_ACC,), jnp.float32),
        pltpu.SemaphoreType.DMA,
        pltpu.SemaphoreType.DMA,
        pltpu.SemaphoreType.DMA,
        pltpu.SemaphoreType.DMA,
    ],
)(_sc_body)


TC_TILES = 8
TC_BLOCK = TC_TILES * CHUNK
TC_GRID = -(-N_ACC // TC_BLOCK)


def _tc_body(alpha_ref, aggP_ref, degf_ref, w1_ref, w2_ref, out_ref):
    i = pl.program_id(0)
    a0 = alpha_ref[0]
    a1 = alpha_ref[1]
    m = jnp.maximum(a0, a1)
    e0 = jnp.exp(a0 - m)
    e1 = jnp.exp(a1 - m)
    inv = 1.0 / (e0 + e1)
    wc = (e0 * inv) * w1_ref[...] + (e1 * inv) * w2_ref[...]
    agg = aggP_ref[0] + aggP_ref[1]
    row_i = lax.broadcasted_iota(jnp.int32, (CHUNK, CHUNK), 0)
    col_i = lax.broadcasted_iota(jnp.int32, (CHUNK, CHUNK), 1)
    is_diag = row_i == col_i
    for t in range(TC_TILES):
        deg = degf_ref[pl.ds(i * TC_BLOCK + t * CHUNK, CHUNK)]
        for w in range(1, NW):
            deg = deg + degf_ref[pl.ds(w * N_ACC + i * TC_BLOCK + t * CHUNK,
                                       CHUNK)]
        scale = 1.0 / jnp.maximum(deg, 1.0)
        srow = jnp.broadcast_to(scale.reshape(1, CHUNK), (CHUNK, CHUNK))
        diag = jnp.where(is_diag, srow, 0.0)
        sub = agg[t * CHUNK:(t + 1) * CHUNK, :]
        aggn = jnp.dot(diag, sub, preferred_element_type=jnp.float32)
        out_ref[t * CHUNK:(t + 1) * CHUNK, :] = jnp.dot(
            aggn, wc, preferred_element_type=jnp.float32)


def _tc_finalize(alpha, agg_p, deg_p, w1, w2):
    return pl.pallas_call(
        _tc_body,
        grid=(TC_GRID,),
        in_specs=[
            pl.BlockSpec(memory_space=pltpu.SMEM),
            pl.BlockSpec((NC, TC_BLOCK, D), lambda i: (0, i, 0)),
            pl.BlockSpec((NW * N_ACC + TC_BLOCK,), lambda i: (0,)),
            pl.BlockSpec((D, D), lambda i: (0, 0)),
            pl.BlockSpec((D, D), lambda i: (0, 0)),
        ],
        out_specs=pl.BlockSpec((TC_BLOCK, D), lambda i: (i, 0)),
        out_shape=jax.ShapeDtypeStruct((N_NODES, D), jnp.float32),
    )(alpha, agg_p, deg_p, w1, w2)


def kernel(x, edge_index, alpha, W1, W2):
    src = edge_index[0]
    dst = edge_index[1]
    pad = E_PAD - N_EDGES
    pad_ar = jnp.arange(pad, dtype=jnp.int32)
    src_p = jnp.concatenate([src, pad_ar % N_NODES])
    dst_p = jnp.concatenate([dst, N_NODES + pad_ar % N_JUNK])
    src3 = src_p.reshape(NW, CHUNKS_PER_TILE, CHUNK)
    dst3 = dst_p.reshape(NW, CHUNKS_PER_TILE, CHUNK)
    agg_p, deg_p = _sc_aggregate(x, src3, dst3)
    return _tc_finalize(alpha, agg_p, deg_p, W1, W2)


# speedup vs baseline: 1.0012x; 1.0012x over previous
import functools

import jax
import jax.numpy as jnp
from jax import lax
from jax.experimental import pallas as pl
from jax.experimental.pallas import tpu as pltpu
from jax.experimental.pallas import tpu_sc as plsc

N_NODES = 10000
N_EDGES = 320000
D = 128

NC = 2
NS = 16
NW = NC * NS
CHUNK = 128
GROUP = 16
GROUPS_PER_TILE = 5
CHUNKS_PER_TILE = GROUP * GROUPS_PER_TILE
E_PAD = NW * CHUNKS_PER_TILE * CHUNK
N_ACC = 10112
N_JUNK = N_ACC - N_NODES
ROWS_PER_TILE = N_ACC // NS
DROWS = N_ACC // CHUNK


def _sc_body(x_hbm, src_hbm, dst_hbm,
             agg_out, deg_out,
             agg_sh, src_v, dst_v, rows_a, rows_b, deg_v,
             gsem_a, gsem_b):
    c = lax.axis_index("c")
    s = lax.axis_index("s")
    wid = c * NS + s

    zero16 = jnp.zeros((16,), jnp.float32)

    def zrow(r, carry):
        for k in range(CHUNK // 16):
            deg_v[pl.ds(r * CHUNK + k * 16, 16)] = zero16
        return carry

    lax.fori_loop(0, DROWS, zrow, 0)

    def zbuf(r, carry):
        for k in range(CHUNK // 16):
            rows_a[r, pl.ds(k * 16, 16)] = zero16
        return carry

    lax.fori_loop(0, CHUNK, zbuf, 0)
    for q in range(4):
        pltpu.sync_copy(
            rows_a,
            agg_sh.at[pl.ds(s * ROWS_PER_TILE + q * CHUNK, CHUNK)])
    pltpu.sync_copy(
        rows_a.at[pl.ds(0, ROWS_PER_TILE - 4 * CHUNK)],
        agg_sh.at[pl.ds(s * ROWS_PER_TILE + 4 * CHUNK,
                        ROWS_PER_TILE - 4 * CHUNK)])
    plsc.subcore_barrier()

    one16 = jnp.ones((16,), jnp.float32)
    rows_bufs = (rows_a, rows_b)
    gsems = (gsem_a, gsem_b)

    def group(g, carry):
        pltpu.sync_copy(src_hbm.at[wid, pl.ds(g * GROUP, GROUP)], src_v)
        pltpu.sync_copy(dst_hbm.at[wid, pl.ds(g * GROUP, GROUP)], dst_v)
        gathers = [
            pltpu.async_copy(x_hbm.at[src_v.at[0]], rows_bufs[0], gsems[0]),
            pltpu.async_copy(x_hbm.at[src_v.at[1]], rows_bufs[1], gsems[1]),
        ]
        for jj in range(GROUP):
            b = jj % 2
            for k in range(CHUNK // 16):
                dk = dst_v[jj, pl.ds(k * 16, 16)]
                plsc.addupdate_scatter(deg_v, [dk], one16)
            gathers[b].wait()
            pltpu.sync_copy(rows_bufs[b], agg_sh.at[dst_v.at[jj]], add=True)
            if jj + 2 < GROUP:
                gathers[b] = pltpu.async_copy(
                    x_hbm.at[src_v.at[jj + 2]], rows_bufs[b], gsems[b])
        return carry

    lax.fori_loop(0, GROUPS_PER_TILE, group, 0)

    plsc.subcore_barrier()
    pltpu.sync_copy(agg_sh.at[pl.ds(s * ROWS_PER_TILE, ROWS_PER_TILE)],
                    agg_out.at[c, pl.ds(s * ROWS_PER_TILE, ROWS_PER_TILE)])
    pltpu.sync_copy(deg_v, deg_out.at[pl.ds(wid * N_ACC, N_ACC)])


_sc_aggregate = functools.partial(
    pl.kernel,
    out_type=[
        jax.ShapeDtypeStruct((NC, N_ACC, D), jnp.float32),
        jax.ShapeDtypeStruct((NW * N_ACC + 1024,), jnp.float32),
    ],
    mesh=plsc.VectorSubcoreMesh(core_axis_name="c", subcore_axis_name="s"),
    compiler_params=pltpu.CompilerParams(needs_layout_passes=False),
    scratch_types=[
        pltpu.VMEM_SHARED((N_ACC, D), jnp.float32),
        pltpu.VMEM((GROUP, CHUNK), jnp.int32),
        pltpu.VMEM((GROUP, CHUNK), jnp.int32),
        pltpu.VMEM((CHUNK, D), jnp.float32),
        pltpu.VMEM((CHUNK, D), jnp.float32),
        pltpu.VMEM((N_ACC,), jnp.float32),
        pltpu.SemaphoreType.DMA,
        pltpu.SemaphoreType.DMA,
    ],
)(_sc_body)


TC_TILES = 8
TC_BLOCK = TC_TILES * CHUNK
TC_GRID = -(-N_ACC // TC_BLOCK)


def _tc_body(alpha_ref, aggP_ref, degf_ref, w1_ref, w2_ref, out_ref):
    i = pl.program_id(0)
    a0 = alpha_ref[0]
    a1 = alpha_ref[1]
    m = jnp.maximum(a0, a1)
    e0 = jnp.exp(a0 - m)
    e1 = jnp.exp(a1 - m)
    inv = 1.0 / (e0 + e1)
    wc = (e0 * inv) * w1_ref[...] + (e1 * inv) * w2_ref[...]
    agg = aggP_ref[0] + aggP_ref[1]
    row_i = lax.broadcasted_iota(jnp.int32, (CHUNK, CHUNK), 0)
    col_i = lax.broadcasted_iota(jnp.int32, (CHUNK, CHUNK), 1)
    is_diag = row_i == col_i
    for t in range(TC_TILES):
        deg = degf_ref[pl.ds(i * TC_BLOCK + t * CHUNK, CHUNK)]
        for w in range(1, NW):
            deg = deg + degf_ref[pl.ds(w * N_ACC + i * TC_BLOCK + t * CHUNK,
                                       CHUNK)]
        scale = 1.0 / jnp.maximum(deg, 1.0)
        srow = jnp.broadcast_to(scale.reshape(1, CHUNK), (CHUNK, CHUNK))
        diag = jnp.where(is_diag, srow, 0.0)
        sub = agg[t * CHUNK:(t + 1) * CHUNK, :]
        aggn = jnp.dot(diag, sub, preferred_element_type=jnp.float32)
        out_ref[t * CHUNK:(t + 1) * CHUNK, :] = jnp.dot(
            aggn, wc, preferred_element_type=jnp.float32)


def _tc_finalize(alpha, agg_p, deg_p, w1, w2):
    return pl.pallas_call(
        _tc_body,
        grid=(TC_GRID,),
        in_specs=[
            pl.BlockSpec(memory_space=pltpu.SMEM),
            pl.BlockSpec((NC, TC_BLOCK, D), lambda i: (0, i, 0)),
            pl.BlockSpec((NW * N_ACC + TC_BLOCK,), lambda i: (0,)),
            pl.BlockSpec((D, D), lambda i: (0, 0)),
            pl.BlockSpec((D, D), lambda i: (0, 0)),
        ],
        out_specs=pl.BlockSpec((TC_BLOCK, D), lambda i: (i, 0)),
        out_shape=jax.ShapeDtypeStruct((N_NODES, D), jnp.float32),
    )(alpha, agg_p, deg_p, w1, w2)


def kernel(x, edge_index, alpha, W1, W2):
    src = edge_index[0]
    dst = edge_index[1]
    pad = E_PAD - N_EDGES
    pad_ar = jnp.arange(pad, dtype=jnp.int32)
    src_p = jnp.concatenate([src, pad_ar % N_NODES])
    dst_p = jnp.concatenate([dst, N_NODES + pad_ar % N_JUNK])
    src3 = src_p.reshape(NW, CHUNKS_PER_TILE, CHUNK)
    dst3 = dst_p.reshape(NW, CHUNKS_PER_TILE, CHUNK)
    agg_p, deg_p = _sc_aggregate(x, src3, dst3)
    return _tc_finalize(alpha, agg_p, deg_p, W1, W2)
